# baseline (speedup 1.0000x reference)
import functools

import jax
import jax.numpy as jnp
from jax import lax
from jax.experimental import pallas as pl
from jax.experimental.pallas import tpu as pltpu
from jax.experimental.pallas import tpu_sc as plsc

_B = 16384
_D = 4096
_NW = 32
_S = 10240
_BPW = _S // _NW
_CH = 8
_NCHUNK = _BPW // _CH

_RPS = 8
_NTC = _B - _S


def _sc_kernel(x_hbm, table_hbm, out_hbm,
               idx_v, rows0, rows1, rows2, gs0, gs1, gs2, os0, os1, os2):
    nc = 2
    wid = lax.axis_index("s") * nc + lax.axis_index("c")
    base = wid * _BPW
    pltpu.sync_copy(x_hbm.at[pl.ds(base, _BPW)], idx_v)

    bufs = (rows0, rows1, rows2)
    gsem = (gs0, gs1, gs2)
    osem = (os0, os1, os2)

    def gstart(c, b):
        pltpu.make_async_copy(
            table_hbm.at[idx_v.at[pl.ds(c * _CH, _CH)]], bufs[b], gsem[b]
        ).start()

    def gwait(b):
        pltpu.make_async_copy(
            table_hbm.at[idx_v.at[pl.ds(0, _CH)]], bufs[b], gsem[b]
        ).wait()

    def ostart(c, b):
        pltpu.make_async_copy(
            bufs[b], out_hbm.at[pl.ds(base + c * _CH, _CH)], osem[b]
        ).start()

    def owait(b):
        pltpu.make_async_copy(
            bufs[b], out_hbm.at[pl.ds(base, _CH)], osem[b]
        ).wait()

    gstart(0, 0)
    gstart(1, 1)
    gwait(0)
    gstart(2, 2)
    ostart(0, 0)
    gwait(1)
    owait(0)
    gstart(3, 0)
    ostart(1, 1)

    def body(r, _):
        for j in range(3):
            c = 2 + 3 * r + j
            s = (2 + j) % 3
            s2 = (1 + j) % 3
            gwait(s)
            owait(s2)
            gstart(c + 2, s2)
            ostart(c, s)
        return ()

    lax.fori_loop(0, (_NCHUNK - 4) // 3, body, ())

    gwait(2)
    owait(1)
    ostart(_NCHUNK - 2, 2)
    gwait(0)
    ostart(_NCHUNK - 1, 0)
    owait(2)
    owait(0)


def _sc_embed(x_flat, weights):
    mesh = plsc.VectorSubcoreMesh(core_axis_name="c", subcore_axis_name="s")
    run = functools.partial(
        pl.kernel,
        mesh=mesh,
        out_type=jax.ShapeDtypeStruct((_S, _D), jnp.float32),
        scratch_types=[
            pltpu.VMEM((_BPW,), jnp.int32),
            pltpu.VMEM((_CH, _D), jnp.float32),
            pltpu.VMEM((_CH, _D), jnp.float32),
            pltpu.VMEM((_CH, _D), jnp.float32),
            pltpu.SemaphoreType.DMA,
            pltpu.SemaphoreType.DMA,
            pltpu.SemaphoreType.DMA,
            pltpu.SemaphoreType.DMA,
            pltpu.SemaphoreType.DMA,
            pltpu.SemaphoreType.DMA,
        ],
    )(_sc_kernel)
    return run(x_flat, weights)


def _tc_body(idx_ref, *refs):
    out_ref = refs[-1]
    for j in range(_RPS):
        out_ref[pl.ds(j, 1), :] = refs[j][0]


def _tc_embed(idx_tc, weights):
    weights3 = weights.reshape(weights.shape[0], 1, _D)
    grid_spec = pltpu.PrefetchScalarGridSpec(
        num_scalar_prefetch=1,
        grid=(_NTC // _RPS,),
        in_specs=[
            pl.BlockSpec((1, 1, _D),
                         functools.partial(
                             lambda j, i, idx: (idx[_RPS * i + j], 0, 0), j))
            for j in range(_RPS)
        ],
        out_specs=pl.BlockSpec((_RPS, _D), lambda i, idx: (i, 0)),
    )
    return pl.pallas_call(
        _tc_body,
        grid_spec=grid_spec,
        out_shape=jax.ShapeDtypeStruct((_NTC, _D), jnp.float32),
    )(idx_tc, *([weights3] * _RPS))


@jax.jit
def _embed(x_flat, weights):
    out_sc = _sc_embed(x_flat, weights)
    out_tc = _tc_embed(x_flat[_S:], weights)
    return jnp.concatenate([out_sc, out_tc], axis=0)


def kernel(x, weights):
    out = _embed(x.reshape(-1), weights)
    return out.reshape(x.shape + (weights.shape[1],))


# speedup vs baseline: 1.0613x; 1.0613x over previous
import functools

import jax
import jax.numpy as jnp
from jax import lax
from jax.experimental import pallas as pl
from jax.experimental.pallas import tpu as pltpu
from jax.experimental.pallas import tpu_sc as plsc

_B = 16384
_D = 4096
_NW = 32
_S = 10240
_BPW = _S // _NW
_CH = 8
_NCHUNK = _BPW // _CH

_RPS = 8
_NTC = _B - _S


def _sc_kernel(x_hbm, table_hbm, out_hbm,
               idx_v, rows0, rows1, rows2, gs0, gs1, gs2, os0, os1, os2):
    nc = 2
    wid = lax.axis_index("s") * nc + lax.axis_index("c")
    base = wid * _BPW
    pltpu.sync_copy(x_hbm.at[pl.ds(base, _BPW)], idx_v)

    bufs = (rows0, rows1, rows2)
    gsem = (gs0, gs1, gs2)
    osem = (os0, os1, os2)

    def gstart(c, b):
        pltpu.make_async_copy(
            table_hbm.at[idx_v.at[pl.ds(c * _CH, _CH)]], bufs[b], gsem[b]
        ).start()

    def gwait(b):
        pltpu.make_async_copy(
            table_hbm.at[idx_v.at[pl.ds(0, _CH)]], bufs[b], gsem[b]
        ).wait()

    def ostart(c, b):
        pltpu.make_async_copy(
            bufs[b], out_hbm.at[pl.ds(base + c * _CH, _CH)], osem[b]
        ).start()

    def owait(b):
        pltpu.make_async_copy(
            bufs[b], out_hbm.at[pl.ds(base, _CH)], osem[b]
        ).wait()

    gstart(0, 0)
    gstart(1, 1)
    gwait(0)
    gstart(2, 2)
    ostart(0, 0)
    gwait(1)
    owait(0)
    gstart(3, 0)
    ostart(1, 1)

    def body(r, _):
        for j in range(3):
            c = 2 + 3 * r + j
            s = (2 + j) % 3
            s2 = (1 + j) % 3
            gwait(s)
            owait(s2)
            gstart(c + 2, s2)
            ostart(c, s)
        return ()

    lax.fori_loop(0, (_NCHUNK - 4) // 3, body, ())

    gwait(2)
    owait(1)
    ostart(_NCHUNK - 2, 2)
    gwait(0)
    ostart(_NCHUNK - 1, 0)
    owait(2)
    owait(0)


def _sc_embed(x_flat, weights):
    mesh = plsc.VectorSubcoreMesh(core_axis_name="c", subcore_axis_name="s")
    run = functools.partial(
        pl.kernel,
        mesh=mesh,
        out_type=jax.ShapeDtypeStruct((_S, _D), jnp.float32),
        scratch_types=[
            pltpu.VMEM((_BPW,), jnp.int32),
            pltpu.VMEM((_CH, _D), jnp.float32),
            pltpu.VMEM((_CH, _D), jnp.float32),
            pltpu.VMEM((_CH, _D), jnp.float32),
            pltpu.SemaphoreType.DMA,
            pltpu.SemaphoreType.DMA,
            pltpu.SemaphoreType.DMA,
            pltpu.SemaphoreType.DMA,
            pltpu.SemaphoreType.DMA,
            pltpu.SemaphoreType.DMA,
        ],
    )(_sc_kernel)
    return run(x_flat, weights)


def _tc_body(idx_ref, *refs):
    out_ref = refs[-1]
    for j in range(_RPS):
        out_ref[pl.ds(j, 1)] = refs[j][...]


def _tc_embed(idx_tc, weights):
    weights3 = weights.reshape(weights.shape[0], 32, 128)
    grid_spec = pltpu.PrefetchScalarGridSpec(
        num_scalar_prefetch=1,
        grid=(_NTC // _RPS,),
        in_specs=[
            pl.BlockSpec((1, 32, 128),
                         functools.partial(
                             lambda j, i, idx: (idx[_RPS * i + j], 0, 0), j))
            for j in range(_RPS)
        ],
        out_specs=pl.BlockSpec((_RPS, 32, 128), lambda i, idx: (i, 0, 0)),
    )
    out = pl.pallas_call(
        _tc_body,
        grid_spec=grid_spec,
        out_shape=jax.ShapeDtypeStruct((_NTC, 32, 128), jnp.float32),
    )(idx_tc, *([weights3] * _RPS))
    return out.reshape(_NTC, _D)


@jax.jit
def _embed(x_flat, weights):
    out_sc = _sc_embed(x_flat, weights)
    out_tc = _tc_embed(x_flat[_S:], weights)
    return jnp.concatenate([out_sc, out_tc], axis=0)


def kernel(x, weights):
    out = _embed(x.reshape(-1), weights)
    return out.reshape(x.shape + (weights.shape[1],))


# speedup vs baseline: 6.0283x; 5.6803x over previous
import functools

import jax
import jax.numpy as jnp
from jax import lax
from jax.experimental import pallas as pl
from jax.experimental.pallas import tpu as pltpu
from jax.experimental.pallas import tpu_sc as plsc

_B = 16384
_D = 4096
_NW = 32
_BPW = _B // _NW
_CH = 8
_NCHUNK = _BPW // _CH
_NBUF = 3


def _embed_kernel(x_hbm, table_hbm, out_hbm,
                  idx_v, rows0, rows1, rows2, gs0, gs1, gs2, os0, os1, os2):
    nc = 2
    wid = lax.axis_index("s") * nc + lax.axis_index("c")
    base = wid * _BPW
    pltpu.sync_copy(x_hbm.at[pl.ds(base, _BPW)], idx_v)

    bufs = (rows0, rows1, rows2)
    gsem = (gs0, gs1, gs2)
    osem = (os0, os1, os2)

    def gstart(c, b):
        pltpu.make_async_copy(
            table_hbm.at[idx_v.at[pl.ds(c * _CH, _CH)]], bufs[b], gsem[b]
        ).start()

    def gwait(b):
        pltpu.make_async_copy(
            table_hbm.at[idx_v.at[pl.ds(0, _CH)]], bufs[b], gsem[b]
        ).wait()

    def ostart(c, b):
        pltpu.make_async_copy(
            bufs[b], out_hbm.at[pl.ds(base + c * _CH, _CH)], osem[b]
        ).start()

    def owait(b):
        pltpu.make_async_copy(
            bufs[b], out_hbm.at[pl.ds(base, _CH)], osem[b]
        ).wait()

    gstart(0, 0)
    gstart(1, 1)
    gwait(0)
    gstart(2, 2)
    ostart(0, 0)
    gwait(1)
    owait(0)
    gstart(3, 0)
    ostart(1, 1)

    def body(r, _):
        for j in range(3):
            c = 2 + 3 * r + j
            s = (2 + j) % 3
            s2 = (1 + j) % 3
            gwait(s)
            owait(s2)
            gstart(c + 2, s2)
            ostart(c, s)
        return ()

    lax.fori_loop(0, (_NCHUNK - 4) // 3, body, ())

    gwait(2)
    owait(1)
    ostart(_NCHUNK - 2, 2)
    gwait(0)
    ostart(_NCHUNK - 1, 0)
    owait(2)
    owait(0)


@jax.jit
def _embed(x_flat, weights):
    mesh = plsc.VectorSubcoreMesh(core_axis_name="c", subcore_axis_name="s")
    run = functools.partial(
        pl.kernel,
        mesh=mesh,
        out_type=jax.ShapeDtypeStruct((_B, _D), jnp.float32),
        scratch_types=[
            pltpu.VMEM((_BPW,), jnp.int32),
            pltpu.VMEM((_CH, _D), jnp.float32),
            pltpu.VMEM((_CH, _D), jnp.float32),
            pltpu.VMEM((_CH, _D), jnp.float32),
            pltpu.SemaphoreType.DMA,
            pltpu.SemaphoreType.DMA,
            pltpu.SemaphoreType.DMA,
            pltpu.SemaphoreType.DMA,
            pltpu.SemaphoreType.DMA,
            pltpu.SemaphoreType.DMA,
        ],
    )(_embed_kernel)
    return run(x_flat, weights)


def kernel(x, weights):
    out = _embed(x.reshape(-1), weights)
    return out.reshape(x.shape + (weights.shape[1],))
